# baseline (speedup 1.0000x reference)
import functools

import jax
import jax.numpy as jnp
from jax import lax
from jax.experimental import pallas as pl
from jax.experimental.pallas import tpu as pltpu
from jax.experimental.pallas import tpu_sc as plsc

B, L, D = 4096, 50, 64
V = 100000
EPS = 1e-5
N = B * L
NC, NS = 2, 16
NW = NC * NS
C = 128
ROWS_W = N // NW
NCH = ROWS_W // C
NG = N // C
GPL = B // C
CP = C + 1


def _stats_body(p_ref, w_ref, g_ref, b_ref, out_ref):
    p = p_ref[...]
    s1 = jnp.sum(p)
    s2 = jnp.sum(p * p)
    mean = s1 / N
    var = s2 / N - mean * mean
    w = w_ref[...]
    scale = w * g_ref[...] * lax.rsqrt(w * w * var + EPS)
    off = b_ref[...] - mean * scale
    out_ref[...] = jnp.concatenate([scale, off], axis=0)


def _price_affine(price2d, w, gamma, beta):
    return pl.pallas_call(
        _stats_body,
        out_shape=jax.ShapeDtypeStruct((2, D), jnp.float32),
    )(price2d, w, gamma, beta)


def _sc_body(brand, material, author, color, price, so_tbl, tf0,
             t_brand, t_material, t_author, t_color, out,
             idx_v, x_v, so_v, tf_v, acc0, acc1, tb0, tb1,
             g0, g1, o0, o1):
    wid = lax.axis_index("s") * NC + lax.axis_index("c")
    r0 = wid * ROWS_W
    tables = (t_brand, t_material, t_author, t_color)
    accs = (acc0, acc1)
    tbs = (tb0, tb1)
    gsems = (g0, g1)
    osems = (o0, o1)

    pltpu.sync_copy(brand.at[pl.ds(r0, ROWS_W)], idx_v.at[0])
    pltpu.sync_copy(material.at[pl.ds(r0, ROWS_W)], idx_v.at[1])
    pltpu.sync_copy(author.at[pl.ds(r0, ROWS_W)], idx_v.at[2])
    pltpu.sync_copy(color.at[pl.ds(r0, ROWS_W)], idx_v.at[3])
    pltpu.sync_copy(price.at[pl.ds(r0, ROWS_W)], x_v)
    pltpu.sync_copy(so_tbl, so_v)
    pltpu.sync_copy(tf0, tf_v)

    sob = [(so_v[0, pl.ds(b * 16, 16)], so_v[1, pl.ds(b * 16, 16)])
           for b in range(4)]
    didx = [jnp.arange(16, dtype=jnp.int32) + blk * 16 for blk in range(4)]

    def price_fill(c, acc):
        cb = c * C

        def row(j, carry):
            j16 = jnp.full((16,), cb + j, jnp.int32)
            xs = plsc.load_gather(x_v, [j16])
            for blk in range(4):
                s, o = carry[2 * blk], carry[2 * blk + 1]
                acc[j, pl.ds(blk * 16, 16)] = jnp.maximum(xs * s + o, 0.0)
            return carry

        carry0 = tuple(v for pair in sob for v in pair)
        lax.fori_loop(0, C, row, carry0)

    def fire_gathers(c, slot):
        cb = c * C
        return [pltpu.async_copy(
            tables[f].at[idx_v.at[f, pl.ds(cb, C)]], accs[slot],
            gsems[slot], add=True) for f in range(4)]

    def wait_gathers(c, slot):
        cb = c * C
        for f in range(4):
            pltpu.make_async_copy(
                tables[f].at[idx_v.at[f, pl.ds(cb, C)]], accs[slot],
                gsems[slot]).wait()

    def fixup(c, acc):
        cb = c * C

        def grp(g, _):
            gb = cb + g * 16
            iv = [idx_v[f, pl.ds(gb, 16)] for f in range(4)]
            bad = ((iv[0] == 0) | (iv[1] == 0) | (iv[2] == 0)
                   | (iv[3] == 0))

            def dofix():
                def rr(j, _):
                    j16 = jnp.full((16,), gb + j, jnp.int32)
                    zs = []
                    for f in range(4):
                        f16 = jnp.full((16,), f, jnp.int32)
                        zf = plsc.load_gather(idx_v, [f16, j16])
                        zs.append(jnp.where(zf == 0, -1.0, 0.0))
                    lr = g * 16 + j
                    for blk in range(4):
                        dsl = pl.ds(blk * 16, 16)
                        corr = (zs[0] * tf_v[0, dsl] + zs[1] * tf_v[1, dsl]
                                + zs[2] * tf_v[2, dsl] + zs[3] * tf_v[3, dsl])
                        plsc.addupdate(acc.at[lr, dsl], corr)
                    return _

                lax.fori_loop(0, 16, rr, None)

            lax.cond(jnp.any(bad), dofix, lambda: None)
            return _

        lax.fori_loop(0, C // 16, grp, None)

    def transpose_chunk(acc, tb):
        def row(j, _):
            j16 = jnp.full((16,), j, jnp.int32)
            for blk in range(4):
                v = acc[j, pl.ds(blk * 16, 16)]
                plsc.store_scatter(tb, [didx[blk], j16], v)
            return _

        lax.fori_loop(0, C, row, None)

    def out_slice(c):
        g = wid * NCH + c
        lg = g // GPL
        b0 = (g % GPL) * C
        return out.at[lg, :, pl.ds(b0, C)]

    def fire_out(c, slot):
        return pltpu.async_copy(tbs[slot].at[:, pl.ds(0, C)], out_slice(c),
                                osems[slot])

    def wait_out(c, slot):
        pltpu.make_async_copy(tbs[slot].at[:, pl.ds(0, C)], out_slice(c),
                              osems[slot]).wait()

    price_fill(0, accs[0])
    fire_gathers(0, 0)

    def S(c, p, q):
        @pl.when(c + 1 < NCH)
        def _issue():
            price_fill(c + 1, accs[p])
            fire_gathers(c + 1, p)

        wait_gathers(c, q)
        fixup(c, accs[q])

        @pl.when(c >= 2)
        def _w():
            wait_out(c - 2, q)
        transpose_chunk(accs[q], tbs[q])
        fire_out(c, q)

    def pair(i, _):
        c = 2 * i
        S(c, 1, 0)
        S(c + 1, 0, 1)
        return _

    lax.fori_loop(0, NCH // 2, pair, None)

    wait_out(NCH - 2, 0)
    wait_out(NCH - 1, 1)


_sc_call = functools.partial(
    pl.kernel,
    out_type=jax.ShapeDtypeStruct((L, D, B), jnp.float32),
    mesh=plsc.VectorSubcoreMesh(core_axis_name="c", subcore_axis_name="s"),
    compiler_params=pltpu.CompilerParams(
        needs_layout_passes=False, use_tc_tiling_on_sc=False),
    scratch_types=[
        pltpu.VMEM((4, ROWS_W), jnp.int32),
        pltpu.VMEM((ROWS_W,), jnp.float32),
        pltpu.VMEM((2, D), jnp.float32),
        pltpu.VMEM((4, D), jnp.float32),
        pltpu.VMEM((C, D), jnp.float32),
        pltpu.VMEM((C, D), jnp.float32),
        pltpu.VMEM((D, CP), jnp.float32),
        pltpu.VMEM((D, CP), jnp.float32),
        pltpu.SemaphoreType.DMA,
        pltpu.SemaphoreType.DMA,
        pltpu.SemaphoreType.DMA,
        pltpu.SemaphoreType.DMA,
    ],
)


def kernel(brand, material, author, color, price, W_price, bn_gamma, bn_beta,
           brand_table, material_table, author_table, color_table):
    bl = jnp.swapaxes(brand, 0, 1).reshape(N)
    ml = jnp.swapaxes(material, 0, 1).reshape(N)
    al = jnp.swapaxes(author, 0, 1).reshape(N)
    cl = jnp.swapaxes(color, 0, 1).reshape(N)
    xl = jnp.swapaxes(price[:, :, 0], 0, 1).reshape(N)
    so_tbl = _price_affine(xl.reshape(NG, C), W_price,
                           bn_gamma.reshape(1, D), bn_beta.reshape(1, D))
    tf0 = jnp.stack([brand_table[0], material_table[0],
                     author_table[0], color_table[0]])
    one = bn_gamma[:1] / bn_gamma[:1]
    bt = brand_table * one
    mt = material_table * one
    at_ = author_table * one
    ct = color_table * one
    sc = _sc_call(_sc_body)
    out = sc(bl, ml, al, cl, xl, so_tbl, tf0, bt, mt, at_, ct)
    return jnp.transpose(out, (2, 0, 1))


# speedup vs baseline: 1.2197x; 1.2197x over previous
import functools

import jax
import jax.numpy as jnp
from jax import lax
from jax.experimental import pallas as pl
from jax.experimental.pallas import tpu as pltpu
from jax.experimental.pallas import tpu_sc as plsc

B, L, D = 4096, 50, 64
V = 100000
EPS = 1e-5
N = B * L
NC, NS = 2, 16
NW = NC * NS
C = 128
ROWS_W = N // NW
NCH = ROWS_W // C
NG = N // C
GPL = B // C
CP = C + 1


def _stats_body(p_ref, w_ref, g_ref, b_ref, bi_ref, mi_ref, ai_ref, ci_ref,
                out_ref, bo_ref, mo_ref, ao_ref, co_ref):
    p = p_ref[...]
    s1 = jnp.sum(p)
    s2 = jnp.sum(p * p)
    mean = s1 / N
    var = s2 / N - mean * mean
    w = w_ref[...]
    scale = w * g_ref[...] * lax.rsqrt(w * w * var + EPS)
    off = b_ref[...] - mean * scale
    out_ref[...] = jnp.concatenate([scale, off], axis=0)
    bo_ref[...] = bi_ref[...].reshape(NG, C)
    mo_ref[...] = mi_ref[...].reshape(NG, C)
    ao_ref[...] = ai_ref[...].reshape(NG, C)
    co_ref[...] = ci_ref[...].reshape(NG, C)


def _price_affine(price2d, w, gamma, beta, bi, mi, ai, ci):
    idx2 = jax.ShapeDtypeStruct((NG, C), jnp.int32)
    return pl.pallas_call(
        _stats_body,
        out_shape=(jax.ShapeDtypeStruct((2, D), jnp.float32),
                   idx2, idx2, idx2, idx2),
    )(price2d, w, gamma, beta, bi, mi, ai, ci)


def _sc_body(brand, material, author, color, price, so_tbl, tf0,
             t_brand, t_material, t_author, t_color, out,
             idx_v, x_v, so_v, tf_v, acc0, acc1, tb0, tb1,
             g0, g1, o0, o1):
    wid = lax.axis_index("s") * NC + lax.axis_index("c")
    r0 = wid * ROWS_W
    tables = (t_brand, t_material, t_author, t_color)
    accs = (acc0, acc1)
    tbs = (tb0, tb1)
    gsems = (g0, g1)
    osems = (o0, o1)

    pltpu.sync_copy(brand.at[pl.ds(r0, ROWS_W)], idx_v.at[0])
    pltpu.sync_copy(material.at[pl.ds(r0, ROWS_W)], idx_v.at[1])
    pltpu.sync_copy(author.at[pl.ds(r0, ROWS_W)], idx_v.at[2])
    pltpu.sync_copy(color.at[pl.ds(r0, ROWS_W)], idx_v.at[3])
    pltpu.sync_copy(price.at[pl.ds(r0, ROWS_W)], x_v)
    pltpu.sync_copy(so_tbl, so_v)
    pltpu.sync_copy(tf0, tf_v)

    sob = [(so_v[0, pl.ds(b * 16, 16)], so_v[1, pl.ds(b * 16, 16)])
           for b in range(4)]
    didx = [jnp.arange(16, dtype=jnp.int32) + blk * 16 for blk in range(4)]

    def price_fill(c, acc):
        cb = c * C

        def row(j, carry):
            j16 = jnp.full((16,), cb + j, jnp.int32)
            xs = plsc.load_gather(x_v, [j16])
            for blk in range(4):
                s, o = carry[2 * blk], carry[2 * blk + 1]
                acc[j, pl.ds(blk * 16, 16)] = jnp.maximum(xs * s + o, 0.0)
            return carry

        carry0 = tuple(v for pair in sob for v in pair)
        lax.fori_loop(0, C, row, carry0)

    def fire_gathers(c, slot):
        cb = c * C
        return [pltpu.async_copy(
            tables[f].at[idx_v.at[f, pl.ds(cb, C)]], accs[slot],
            gsems[slot], add=True) for f in range(4)]

    def wait_gathers(c, slot):
        cb = c * C
        for f in range(4):
            pltpu.make_async_copy(
                tables[f].at[idx_v.at[f, pl.ds(cb, C)]], accs[slot],
                gsems[slot]).wait()

    def fixup(c, acc):
        cb = c * C

        def grp(g, _):
            gb = cb + g * 16
            iv = [idx_v[f, pl.ds(gb, 16)] for f in range(4)]
            bad = ((iv[0] == 0) | (iv[1] == 0) | (iv[2] == 0)
                   | (iv[3] == 0))

            def dofix():
                def rr(j, _):
                    j16 = jnp.full((16,), gb + j, jnp.int32)
                    zs = []
                    for f in range(4):
                        f16 = jnp.full((16,), f, jnp.int32)
                        zf = plsc.load_gather(idx_v, [f16, j16])
                        zs.append(jnp.where(zf == 0, -1.0, 0.0))
                    lr = g * 16 + j
                    for blk in range(4):
                        dsl = pl.ds(blk * 16, 16)
                        corr = (zs[0] * tf_v[0, dsl] + zs[1] * tf_v[1, dsl]
                                + zs[2] * tf_v[2, dsl] + zs[3] * tf_v[3, dsl])
                        plsc.addupdate(acc.at[lr, dsl], corr)
                    return _

                lax.fori_loop(0, 16, rr, None)

            lax.cond(jnp.any(bad), dofix, lambda: None)
            return _

        lax.fori_loop(0, C // 16, grp, None)

    def transpose_chunk(acc, tb):
        def row(j, _):
            j16 = jnp.full((16,), j, jnp.int32)
            for blk in range(4):
                v = acc[j, pl.ds(blk * 16, 16)]
                plsc.store_scatter(tb, [didx[blk], j16], v)
            return _

        lax.fori_loop(0, C, row, None)

    def out_slice(c):
        g = wid * NCH + c
        lg = g // GPL
        b0 = (g % GPL) * C
        return out.at[lg, :, pl.ds(b0, C)]

    def fire_out(c, slot):
        return pltpu.async_copy(tbs[slot].at[:, pl.ds(0, C)], out_slice(c),
                                osems[slot])

    def wait_out(c, slot):
        pltpu.make_async_copy(tbs[slot].at[:, pl.ds(0, C)], out_slice(c),
                              osems[slot]).wait()

    price_fill(0, accs[0])
    fire_gathers(0, 0)

    def S(c, p, q):
        @pl.when(c + 1 < NCH)
        def _issue():
            price_fill(c + 1, accs[p])
            fire_gathers(c + 1, p)

        wait_gathers(c, q)
        fixup(c, accs[q])

        @pl.when(c >= 2)
        def _w():
            wait_out(c - 2, q)
        transpose_chunk(accs[q], tbs[q])
        fire_out(c, q)

    def pair(i, _):
        c = 2 * i
        S(c, 1, 0)
        S(c + 1, 0, 1)
        return _

    lax.fori_loop(0, NCH // 2, pair, None)

    wait_out(NCH - 2, 0)
    wait_out(NCH - 1, 1)


_sc_call = functools.partial(
    pl.kernel,
    out_type=jax.ShapeDtypeStruct((L, D, B), jnp.float32),
    mesh=plsc.VectorSubcoreMesh(core_axis_name="c", subcore_axis_name="s"),
    compiler_params=pltpu.CompilerParams(
        needs_layout_passes=False, use_tc_tiling_on_sc=False),
    scratch_types=[
        pltpu.VMEM((4, ROWS_W), jnp.int32),
        pltpu.VMEM((ROWS_W,), jnp.float32),
        pltpu.VMEM((2, D), jnp.float32),
        pltpu.VMEM((4, D), jnp.float32),
        pltpu.VMEM((C, D), jnp.float32),
        pltpu.VMEM((C, D), jnp.float32),
        pltpu.VMEM((D, CP), jnp.float32),
        pltpu.VMEM((D, CP), jnp.float32),
        pltpu.SemaphoreType.DMA,
        pltpu.SemaphoreType.DMA,
        pltpu.SemaphoreType.DMA,
        pltpu.SemaphoreType.DMA,
    ],
)


def kernel(brand, material, author, color, price, W_price, bn_gamma, bn_beta,
           brand_table, material_table, author_table, color_table):
    xl = jnp.swapaxes(price[:, :, 0], 0, 1).reshape(N)
    so_tbl, b2, m2, a2, c2 = _price_affine(
        xl.reshape(NG, C), W_price,
        bn_gamma.reshape(1, D), bn_beta.reshape(1, D),
        jnp.swapaxes(brand, 0, 1), jnp.swapaxes(material, 0, 1),
        jnp.swapaxes(author, 0, 1), jnp.swapaxes(color, 0, 1))
    bl, ml, al, cl = (b2.reshape(N), m2.reshape(N),
                      a2.reshape(N), c2.reshape(N))
    tf0 = jnp.stack([brand_table[0], material_table[0],
                     author_table[0], color_table[0]])
    sc = _sc_call(_sc_body)
    out = sc(bl, ml, al, cl, xl, so_tbl, tf0,
             brand_table, material_table, author_table, color_table)
    return jnp.transpose(out, (2, 0, 1))


# speedup vs baseline: 1.3878x; 1.1379x over previous
import functools

import jax
import jax.numpy as jnp
from jax import lax
from jax.experimental import pallas as pl
from jax.experimental.pallas import tpu as pltpu
from jax.experimental.pallas import tpu_sc as plsc

B, L, D = 4096, 50, 64
V = 100000
EPS = 1e-5
N = B * L
NC, NS = 2, 16
NW = NC * NS
C = 128
ROWS_W = N // NW
NCH = ROWS_W // C
NG = N // C
GPL = B // C
CP = C + 1


def _stats_body(p_ref, w_ref, g_ref, b_ref, bi_ref, mi_ref, ai_ref, ci_ref,
                out_ref, bo_ref, mo_ref, ao_ref, co_ref):
    p = p_ref[...]
    s1 = jnp.sum(p)
    s2 = jnp.sum(p * p)
    mean = s1 / N
    var = s2 / N - mean * mean
    w = w_ref[...]
    scale = w * g_ref[...] * lax.rsqrt(w * w * var + EPS)
    off = b_ref[...] - mean * scale
    out_ref[...] = jnp.concatenate([scale, off], axis=0)
    bo_ref[...] = bi_ref[...].reshape(NG, C)
    mo_ref[...] = mi_ref[...].reshape(NG, C)
    ao_ref[...] = ai_ref[...].reshape(NG, C)
    co_ref[...] = ci_ref[...].reshape(NG, C)


def _price_affine(price2d, w, gamma, beta, bi, mi, ai, ci):
    idx2 = jax.ShapeDtypeStruct((NG, C), jnp.int32)
    return pl.pallas_call(
        _stats_body,
        out_shape=(jax.ShapeDtypeStruct((2, D), jnp.float32),
                   idx2, idx2, idx2, idx2),
    )(price2d, w, gamma, beta, bi, mi, ai, ci)


def _sc_body(brand, material, author, color, price, so_tbl, tf0,
             t_brand, t_material, t_author, t_color, out,
             idx_v, x_v, so_v, tf_v, acc0, acc1, tb0, tb1,
             g0, g1, o0, o1):
    wid = lax.axis_index("s") * NC + lax.axis_index("c")
    r0 = wid * ROWS_W
    tables = (t_brand, t_material, t_author, t_color)
    accs = (acc0, acc1)
    tbs = (tb0, tb1)
    gsems = (g0, g1)
    osems = (o0, o1)

    pltpu.sync_copy(brand.at[pl.ds(r0, ROWS_W)], idx_v.at[0])
    pltpu.sync_copy(material.at[pl.ds(r0, ROWS_W)], idx_v.at[1])
    pltpu.sync_copy(author.at[pl.ds(r0, ROWS_W)], idx_v.at[2])
    pltpu.sync_copy(color.at[pl.ds(r0, ROWS_W)], idx_v.at[3])
    pltpu.sync_copy(price.at[pl.ds(r0, ROWS_W)], x_v)
    pltpu.sync_copy(so_tbl, so_v)
    pltpu.sync_copy(tf0, tf_v)

    sob = [(so_v[0, pl.ds(b * 16, 16)], so_v[1, pl.ds(b * 16, 16)])
           for b in range(4)]
    didx = [jnp.arange(16, dtype=jnp.int32) + blk * 16 for blk in range(4)]

    def price_fill(c, acc):
        cb = c * C

        def row(j, carry):
            j16 = jnp.full((16,), cb + j, jnp.int32)
            xs = plsc.load_gather(x_v, [j16])
            for blk in range(4):
                s, o = carry[2 * blk], carry[2 * blk + 1]
                acc[j, pl.ds(blk * 16, 16)] = jnp.maximum(xs * s + o, 0.0)
            return carry

        carry0 = tuple(v for pair in sob for v in pair)
        lax.fori_loop(0, C, row, carry0)

    def fire_gathers(c, slot):
        cb = c * C
        return [pltpu.async_copy(
            tables[f].at[idx_v.at[f, pl.ds(cb, C)]], accs[slot],
            gsems[slot], add=True) for f in range(4)]

    def wait_gathers(c, slot):
        cb = c * C
        for f in range(4):
            pltpu.make_async_copy(
                tables[f].at[idx_v.at[f, pl.ds(cb, C)]], accs[slot],
                gsems[slot]).wait()

    def fixup(c, acc):
        cb = c * C

        def grp(g, _):
            gb = cb + g * 16
            iv = [idx_v[f, pl.ds(gb, 16)] for f in range(4)]
            bad = ((iv[0] == 0) | (iv[1] == 0) | (iv[2] == 0)
                   | (iv[3] == 0))

            def dofix():
                def rr(j, _):
                    j16 = jnp.full((16,), gb + j, jnp.int32)
                    zs = []
                    for f in range(4):
                        f16 = jnp.full((16,), f, jnp.int32)
                        zf = plsc.load_gather(idx_v, [f16, j16])
                        zs.append(jnp.where(zf == 0, -1.0, 0.0))
                    lr = g * 16 + j
                    for blk in range(4):
                        dsl = pl.ds(blk * 16, 16)
                        corr = (zs[0] * tf_v[0, dsl] + zs[1] * tf_v[1, dsl]
                                + zs[2] * tf_v[2, dsl] + zs[3] * tf_v[3, dsl])
                        plsc.addupdate(acc.at[lr, dsl], corr)
                    return _

                lax.fori_loop(0, 16, rr, None)

            lax.cond(jnp.any(bad), dofix, lambda: None)
            return _

        lax.fori_loop(0, C // 16, grp, None)

    def transpose_chunk(acc, tb):
        def row(j, _):
            j16 = jnp.full((16,), j, jnp.int32)
            for blk in range(4):
                v = acc[j, pl.ds(blk * 16, 16)]
                plsc.store_scatter(tb, [didx[blk], j16], v)
            return _

        lax.fori_loop(0, C, row, None)

    def out_copies(c, slot, make_only):
        g = wid * NCH + c
        lg = g // GPL
        bt = g % GPL
        for band in range(8):
            tile = (lg * 8 + band) * GPL + bt
            src = tbs[slot].at[pl.ds(band * 8, 8), pl.ds(0, C)]
            dst = out.at[pl.ds(tile * 8, 8), :]
            if make_only:
                pltpu.make_async_copy(src, dst, osems[slot]).wait()
            else:
                pltpu.async_copy(src, dst, osems[slot])

    def fire_out(c, slot):
        out_copies(c, slot, make_only=False)

    def wait_out(c, slot):
        out_copies(c, slot, make_only=True)

    price_fill(0, accs[0])
    fire_gathers(0, 0)

    def S(c, p, q):
        @pl.when(c + 1 < NCH)
        def _issue():
            price_fill(c + 1, accs[p])
            fire_gathers(c + 1, p)

        wait_gathers(c, q)
        fixup(c, accs[q])

        @pl.when(c >= 2)
        def _w():
            wait_out(c - 2, q)
        transpose_chunk(accs[q], tbs[q])
        fire_out(c, q)

    def pair(i, _):
        c = 2 * i
        S(c, 1, 0)
        S(c + 1, 0, 1)
        return _

    lax.fori_loop(0, NCH // 2, pair, None)

    wait_out(NCH - 2, 0)
    wait_out(NCH - 1, 1)


_sc_call = functools.partial(
    pl.kernel,
    out_type=jax.ShapeDtypeStruct((L * D * B // 128, 128), jnp.float32),
    mesh=plsc.VectorSubcoreMesh(core_axis_name="c", subcore_axis_name="s"),
    compiler_params=pltpu.CompilerParams(
        needs_layout_passes=False, use_tc_tiling_on_sc=False),
    scratch_types=[
        pltpu.VMEM((4, ROWS_W), jnp.int32),
        pltpu.VMEM((ROWS_W,), jnp.float32),
        pltpu.VMEM((2, D), jnp.float32),
        pltpu.VMEM((4, D), jnp.float32),
        pltpu.VMEM((C, D), jnp.float32),
        pltpu.VMEM((C, D), jnp.float32),
        pltpu.VMEM((D, CP), jnp.float32),
        pltpu.VMEM((D, CP), jnp.float32),
        pltpu.SemaphoreType.DMA,
        pltpu.SemaphoreType.DMA,
        pltpu.SemaphoreType.DMA,
        pltpu.SemaphoreType.DMA,
    ],
)


def kernel(brand, material, author, color, price, W_price, bn_gamma, bn_beta,
           brand_table, material_table, author_table, color_table):
    xl = jnp.swapaxes(price[:, :, 0], 0, 1).reshape(N)
    so_tbl, b2, m2, a2, c2 = _price_affine(
        xl.reshape(NG, C), W_price,
        bn_gamma.reshape(1, D), bn_beta.reshape(1, D),
        jnp.swapaxes(brand, 0, 1), jnp.swapaxes(material, 0, 1),
        jnp.swapaxes(author, 0, 1), jnp.swapaxes(color, 0, 1))
    bl, ml, al, cl = (b2.reshape(N), m2.reshape(N),
                      a2.reshape(N), c2.reshape(N))
    tf0 = jnp.stack([brand_table[0], material_table[0],
                     author_table[0], color_table[0]])
    sc = _sc_call(_sc_body)
    out = sc(bl, ml, al, cl, xl, so_tbl, tf0,
             brand_table, material_table, author_table, color_table)
    o5 = out.reshape(L, 8, GPL, 8, C)
    return jnp.transpose(o5, (2, 4, 0, 1, 3)).reshape(B, L, D)


# speedup vs baseline: 1.6752x; 1.2071x over previous
import functools

import jax
import jax.numpy as jnp
from jax import lax
from jax.experimental import pallas as pl
from jax.experimental.pallas import tpu as pltpu
from jax.experimental.pallas import tpu_sc as plsc

B, L, D = 4096, 50, 64
V = 100000
EPS = 1e-5
N = B * L
NC, NS = 2, 16
NW = NC * NS
C = 128
ROWS_W = N // NW
NCH = ROWS_W // C
NG = N // C
GPL = B // C
CP = C + 1


def _stats_body(p_ref, w_ref, g_ref, b_ref, bi_ref, mi_ref, ai_ref, ci_ref,
                out_ref, bo_ref, mo_ref, ao_ref, co_ref):
    p = p_ref[...]
    s1 = jnp.sum(p)
    s2 = jnp.sum(p * p)
    mean = s1 / N
    var = s2 / N - mean * mean
    w = w_ref[...]
    scale = w * g_ref[...] * lax.rsqrt(w * w * var + EPS)
    off = b_ref[...] - mean * scale
    out_ref[...] = jnp.concatenate([scale, off], axis=0)
    bo_ref[...] = bi_ref[...].reshape(NG, C)
    mo_ref[...] = mi_ref[...].reshape(NG, C)
    ao_ref[...] = ai_ref[...].reshape(NG, C)
    co_ref[...] = ci_ref[...].reshape(NG, C)


def _price_affine(price2d, w, gamma, beta, bi, mi, ai, ci):
    idx2 = jax.ShapeDtypeStruct((NG, C), jnp.int32)
    return pl.pallas_call(
        _stats_body,
        out_shape=(jax.ShapeDtypeStruct((2, D), jnp.float32),
                   idx2, idx2, idx2, idx2),
    )(price2d, w, gamma, beta, bi, mi, ai, ci)


def _sc_body(brand, material, author, color, price, so_tbl, tf0,
             t_brand, t_material, t_author, t_color, out,
             idx_v, x_v, so_v, tf_v, acc0, acc1, tb0, tb1,
             g0, g1, o0, o1):
    wid = lax.axis_index("s") * NC + lax.axis_index("c")
    r0 = wid * ROWS_W
    tables = (t_brand, t_material, t_author, t_color)
    accs = (acc0, acc1)
    tbs = (tb0, tb1)
    gsems = (g0, g1)
    osems = (o0, o1)

    pltpu.sync_copy(brand.at[pl.ds(r0, ROWS_W)], idx_v.at[0])
    pltpu.sync_copy(material.at[pl.ds(r0, ROWS_W)], idx_v.at[1])
    pltpu.sync_copy(author.at[pl.ds(r0, ROWS_W)], idx_v.at[2])
    pltpu.sync_copy(color.at[pl.ds(r0, ROWS_W)], idx_v.at[3])
    pltpu.sync_copy(price.at[pl.ds(r0, ROWS_W)], x_v)
    pltpu.sync_copy(so_tbl, so_v)
    pltpu.sync_copy(tf0, tf_v)

    sob = [(so_v[0, pl.ds(b * 16, 16)], so_v[1, pl.ds(b * 16, 16)])
           for b in range(4)]
    didx = [jnp.arange(16, dtype=jnp.int32) + blk * 16 for blk in range(4)]

    def price_fill(c, acc):
        cb = c * C

        carry0 = tuple(v for pair in sob for v in pair)

        @plsc.parallel_loop(0, C, unroll=4, carry=carry0)
        def row(j, carry):
            j16 = jnp.full((16,), cb + j, jnp.int32)
            xs = plsc.load_gather(x_v, [j16])
            for blk in range(4):
                s, o = carry[2 * blk], carry[2 * blk + 1]
                acc[j, pl.ds(blk * 16, 16)] = jnp.maximum(xs * s + o, 0.0)
            return carry

    def fire_gathers(c, slot):
        cb = c * C
        return [pltpu.async_copy(
            tables[f].at[idx_v.at[f, pl.ds(cb, C)]], accs[slot],
            gsems[slot], add=True) for f in range(4)]

    def wait_gathers(c, slot):
        cb = c * C
        for f in range(4):
            pltpu.make_async_copy(
                tables[f].at[idx_v.at[f, pl.ds(cb, C)]], accs[slot],
                gsems[slot]).wait()

    def fixup(c, acc):
        cb = c * C

        def grp(g, _):
            gb = cb + g * 16
            iv = [idx_v[f, pl.ds(gb, 16)] for f in range(4)]
            bad = ((iv[0] == 0) | (iv[1] == 0) | (iv[2] == 0)
                   | (iv[3] == 0))

            def dofix():
                def rr(j, _):
                    j16 = jnp.full((16,), gb + j, jnp.int32)
                    zs = []
                    for f in range(4):
                        f16 = jnp.full((16,), f, jnp.int32)
                        zf = plsc.load_gather(idx_v, [f16, j16])
                        zs.append(jnp.where(zf == 0, -1.0, 0.0))
                    lr = g * 16 + j
                    for blk in range(4):
                        dsl = pl.ds(blk * 16, 16)
                        corr = (zs[0] * tf_v[0, dsl] + zs[1] * tf_v[1, dsl]
                                + zs[2] * tf_v[2, dsl] + zs[3] * tf_v[3, dsl])
                        plsc.addupdate(acc.at[lr, dsl], corr)
                    return _

                lax.fori_loop(0, 16, rr, None)

            lax.cond(jnp.any(bad), dofix, lambda: None)
            return _

        lax.fori_loop(0, C // 16, grp, None)

    def transpose_chunk(acc, tb):
        @plsc.parallel_loop(0, C, unroll=4)
        def row(j):
            j16 = jnp.full((16,), j, jnp.int32)
            for blk in range(4):
                v = acc[j, pl.ds(blk * 16, 16)]
                plsc.store_scatter(tb, [didx[blk], j16], v)

    def out_copies(c, slot, make_only):
        g = wid * NCH + c
        lg = g // GPL
        bt = g % GPL
        for band in range(8):
            tile = (lg * 8 + band) * GPL + bt
            src = tbs[slot].at[pl.ds(band * 8, 8), pl.ds(0, C)]
            dst = out.at[pl.ds(tile * 8, 8), :]
            if make_only:
                pltpu.make_async_copy(src, dst, osems[slot]).wait()
            else:
                pltpu.async_copy(src, dst, osems[slot])

    def fire_out(c, slot):
        out_copies(c, slot, make_only=False)

    def wait_out(c, slot):
        out_copies(c, slot, make_only=True)

    price_fill(0, accs[0])
    fire_gathers(0, 0)

    def S(c, p, q):
        @pl.when(c + 1 < NCH)
        def _issue():
            price_fill(c + 1, accs[p])
            fire_gathers(c + 1, p)

        wait_gathers(c, q)
        fixup(c, accs[q])

        @pl.when(c >= 2)
        def _w():
            wait_out(c - 2, q)
        transpose_chunk(accs[q], tbs[q])
        fire_out(c, q)

    def pair(i, _):
        c = 2 * i
        S(c, 1, 0)
        S(c + 1, 0, 1)
        return _

    lax.fori_loop(0, NCH // 2, pair, None)

    wait_out(NCH - 2, 0)
    wait_out(NCH - 1, 1)


_sc_call = functools.partial(
    pl.kernel,
    out_type=jax.ShapeDtypeStruct((L * D * B // 128, 128), jnp.float32),
    mesh=plsc.VectorSubcoreMesh(core_axis_name="c", subcore_axis_name="s"),
    compiler_params=pltpu.CompilerParams(
        needs_layout_passes=False, use_tc_tiling_on_sc=False),
    scratch_types=[
        pltpu.VMEM((4, ROWS_W), jnp.int32),
        pltpu.VMEM((ROWS_W,), jnp.float32),
        pltpu.VMEM((2, D), jnp.float32),
        pltpu.VMEM((4, D), jnp.float32),
        pltpu.VMEM((C, D), jnp.float32),
        pltpu.VMEM((C, D), jnp.float32),
        pltpu.VMEM((D, CP), jnp.float32),
        pltpu.VMEM((D, CP), jnp.float32),
        pltpu.SemaphoreType.DMA,
        pltpu.SemaphoreType.DMA,
        pltpu.SemaphoreType.DMA,
        pltpu.SemaphoreType.DMA,
    ],
)


def kernel(brand, material, author, color, price, W_price, bn_gamma, bn_beta,
           brand_table, material_table, author_table, color_table):
    xl = jnp.swapaxes(price[:, :, 0], 0, 1).reshape(N)
    so_tbl, b2, m2, a2, c2 = _price_affine(
        xl.reshape(NG, C), W_price,
        bn_gamma.reshape(1, D), bn_beta.reshape(1, D),
        jnp.swapaxes(brand, 0, 1), jnp.swapaxes(material, 0, 1),
        jnp.swapaxes(author, 0, 1), jnp.swapaxes(color, 0, 1))
    bl, ml, al, cl = (b2.reshape(N), m2.reshape(N),
                      a2.reshape(N), c2.reshape(N))
    tf0 = jnp.stack([brand_table[0], material_table[0],
                     author_table[0], color_table[0]])
    sc = _sc_call(_sc_body)
    out = sc(bl, ml, al, cl, xl, so_tbl, tf0,
             brand_table, material_table, author_table, color_table)
    o5 = out.reshape(L, 8, GPL, 8, C)
    return jnp.transpose(o5, (2, 4, 0, 1, 3)).reshape(B, L, D)


# speedup vs baseline: 1.6854x; 1.0061x over previous
import functools

import jax
import jax.numpy as jnp
from jax import lax
from jax.experimental import pallas as pl
from jax.experimental.pallas import tpu as pltpu
from jax.experimental.pallas import tpu_sc as plsc

B, L, D = 4096, 50, 64
V = 100000
EPS = 1e-5
N = B * L
NC, NS = 2, 16
NW = NC * NS
C = 128
ROWS_W = N // NW
NCH = ROWS_W // C
NG = N // C
GPL = B // C
CP = C + 1


def _stats_body(p_ref, w_ref, g_ref, b_ref, bi_ref, mi_ref, ai_ref, ci_ref,
                out_ref, bo_ref, mo_ref, ao_ref, co_ref):
    p = p_ref[...]
    s1 = jnp.sum(p)
    s2 = jnp.sum(p * p)
    mean = s1 / N
    var = s2 / N - mean * mean
    w = w_ref[...]
    scale = w * g_ref[...] * lax.rsqrt(w * w * var + EPS)
    off = b_ref[...] - mean * scale
    out_ref[...] = jnp.concatenate([scale, off], axis=0)
    bo_ref[...] = bi_ref[...].reshape(NG, C)
    mo_ref[...] = mi_ref[...].reshape(NG, C)
    ao_ref[...] = ai_ref[...].reshape(NG, C)
    co_ref[...] = ci_ref[...].reshape(NG, C)


def _price_affine(price2d, w, gamma, beta, bi, mi, ai, ci):
    idx2 = jax.ShapeDtypeStruct((NG, C), jnp.int32)
    return pl.pallas_call(
        _stats_body,
        out_shape=(jax.ShapeDtypeStruct((2, D), jnp.float32),
                   idx2, idx2, idx2, idx2),
    )(price2d, w, gamma, beta, bi, mi, ai, ci)


def _sc_body(brand, material, author, color, price, so_tbl, tf0,
             t_brand, t_material, t_author, t_color, out,
             idx_v, x_v, so_v, tf_v, acc0, acc1, tb0, tb1,
             g0, g1, o0, o1):
    wid = lax.axis_index("s") * NC + lax.axis_index("c")
    r0 = wid * ROWS_W
    tables = (t_brand, t_material, t_author, t_color)
    accs = (acc0, acc1)
    tbs = (tb0, tb1)
    gsems = (g0, g1)
    osems = (o0, o1)

    stage = [
        pltpu.async_copy(brand.at[pl.ds(r0, ROWS_W)], idx_v.at[0], g0),
        pltpu.async_copy(material.at[pl.ds(r0, ROWS_W)], idx_v.at[1], g0),
        pltpu.async_copy(author.at[pl.ds(r0, ROWS_W)], idx_v.at[2], g0),
        pltpu.async_copy(color.at[pl.ds(r0, ROWS_W)], idx_v.at[3], g0),
        pltpu.async_copy(price.at[pl.ds(r0, ROWS_W)], x_v, g0),
        pltpu.async_copy(so_tbl, so_v, g0),
        pltpu.async_copy(tf0, tf_v, g0),
    ]
    for cp in stage:
        cp.wait()

    sob = [(so_v[0, pl.ds(b * 16, 16)], so_v[1, pl.ds(b * 16, 16)])
           for b in range(4)]
    didx = [jnp.arange(16, dtype=jnp.int32) + blk * 16 for blk in range(4)]

    def price_fill(c, acc):
        cb = c * C

        carry0 = tuple(v for pair in sob for v in pair)

        @plsc.parallel_loop(0, C, unroll=8, carry=carry0)
        def row(j, carry):
            j16 = jnp.full((16,), cb + j, jnp.int32)
            xs = plsc.load_gather(x_v, [j16])
            for blk in range(4):
                s, o = carry[2 * blk], carry[2 * blk + 1]
                acc[j, pl.ds(blk * 16, 16)] = jnp.maximum(xs * s + o, 0.0)
            return carry

    def fire_gathers(c, slot):
        cb = c * C
        return [pltpu.async_copy(
            tables[f].at[idx_v.at[f, pl.ds(cb, C)]], accs[slot],
            gsems[slot], add=True) for f in range(4)]

    def wait_gathers(c, slot):
        cb = c * C
        for f in range(4):
            pltpu.make_async_copy(
                tables[f].at[idx_v.at[f, pl.ds(cb, C)]], accs[slot],
                gsems[slot]).wait()

    def fixup(c, acc):
        cb = c * C

        def grp(g, _):
            gb = cb + g * 16
            iv = [idx_v[f, pl.ds(gb, 16)] for f in range(4)]
            bad = ((iv[0] == 0) | (iv[1] == 0) | (iv[2] == 0)
                   | (iv[3] == 0))

            def dofix():
                def rr(j, _):
                    j16 = jnp.full((16,), gb + j, jnp.int32)
                    zs = []
                    for f in range(4):
                        f16 = jnp.full((16,), f, jnp.int32)
                        zf = plsc.load_gather(idx_v, [f16, j16])
                        zs.append(jnp.where(zf == 0, -1.0, 0.0))
                    lr = g * 16 + j
                    for blk in range(4):
                        dsl = pl.ds(blk * 16, 16)
                        corr = (zs[0] * tf_v[0, dsl] + zs[1] * tf_v[1, dsl]
                                + zs[2] * tf_v[2, dsl] + zs[3] * tf_v[3, dsl])
                        plsc.addupdate(acc.at[lr, dsl], corr)
                    return _

                lax.fori_loop(0, 16, rr, None)

            lax.cond(jnp.any(bad), dofix, lambda: None)
            return _

        lax.fori_loop(0, C // 16, grp, None)

    def transpose_chunk(acc, tb):
        @plsc.parallel_loop(0, C, unroll=8)
        def row(j):
            j16 = jnp.full((16,), j, jnp.int32)
            for blk in range(4):
                v = acc[j, pl.ds(blk * 16, 16)]
                plsc.store_scatter(tb, [didx[blk], j16], v)

    def out_copies(c, slot, make_only):
        g = wid * NCH + c
        lg = g // GPL
        bt = g % GPL
        for band in range(8):
            tile = (lg * 8 + band) * GPL + bt
            src = tbs[slot].at[pl.ds(band * 8, 8), pl.ds(0, C)]
            dst = out.at[pl.ds(tile * 8, 8), :]
            if make_only:
                pltpu.make_async_copy(src, dst, osems[slot]).wait()
            else:
                pltpu.async_copy(src, dst, osems[slot])

    def fire_out(c, slot):
        out_copies(c, slot, make_only=False)

    def wait_out(c, slot):
        out_copies(c, slot, make_only=True)

    price_fill(0, accs[0])
    fire_gathers(0, 0)

    def S(c, p, q):
        @pl.when(c + 1 < NCH)
        def _issue():
            price_fill(c + 1, accs[p])
            fire_gathers(c + 1, p)

        wait_gathers(c, q)
        fixup(c, accs[q])

        @pl.when(c >= 2)
        def _w():
            wait_out(c - 2, q)
        transpose_chunk(accs[q], tbs[q])
        fire_out(c, q)

    def pair(i, _):
        c = 2 * i
        S(c, 1, 0)
        S(c + 1, 0, 1)
        return _

    lax.fori_loop(0, NCH // 2, pair, None)

    wait_out(NCH - 2, 0)
    wait_out(NCH - 1, 1)


_sc_call = functools.partial(
    pl.kernel,
    out_type=jax.ShapeDtypeStruct((L * D * B // 128, 128), jnp.float32),
    mesh=plsc.VectorSubcoreMesh(core_axis_name="c", subcore_axis_name="s"),
    compiler_params=pltpu.CompilerParams(
        needs_layout_passes=False, use_tc_tiling_on_sc=False),
    scratch_types=[
        pltpu.VMEM((4, ROWS_W), jnp.int32),
        pltpu.VMEM((ROWS_W,), jnp.float32),
        pltpu.VMEM((2, D), jnp.float32),
        pltpu.VMEM((4, D), jnp.float32),
        pltpu.VMEM((C, D), jnp.float32),
        pltpu.VMEM((C, D), jnp.float32),
        pltpu.VMEM((D, CP), jnp.float32),
        pltpu.VMEM((D, CP), jnp.float32),
        pltpu.SemaphoreType.DMA,
        pltpu.SemaphoreType.DMA,
        pltpu.SemaphoreType.DMA,
        pltpu.SemaphoreType.DMA,
    ],
)


def kernel(brand, material, author, color, price, W_price, bn_gamma, bn_beta,
           brand_table, material_table, author_table, color_table):
    xl = jnp.swapaxes(price[:, :, 0], 0, 1).reshape(N)
    so_tbl, b2, m2, a2, c2 = _price_affine(
        xl.reshape(NG, C), W_price,
        bn_gamma.reshape(1, D), bn_beta.reshape(1, D),
        jnp.swapaxes(brand, 0, 1), jnp.swapaxes(material, 0, 1),
        jnp.swapaxes(author, 0, 1), jnp.swapaxes(color, 0, 1))
    bl, ml, al, cl = (b2.reshape(N), m2.reshape(N),
                      a2.reshape(N), c2.reshape(N))
    tf0 = jnp.stack([brand_table[0], material_table[0],
                     author_table[0], color_table[0]])
    sc = _sc_call(_sc_body)
    out = sc(bl, ml, al, cl, xl, so_tbl, tf0,
             brand_table, material_table, author_table, color_table)
    o5 = out.reshape(L, 8, GPL, 8, C)
    return jnp.transpose(o5, (2, 4, 0, 1, 3)).reshape(B, L, D)
